# Initial kernel scaffold; baseline (speedup 1.0000x reference)
import jax
import jax.numpy as jnp
from jax.experimental import pallas as pl


def kernel(x, edge_index, W_in, b_in, W_msg, b_msg, W_ih, W_hh, b_ih, b_hh, W_mu, b_mu, W_lv, b_lv):
    raise NotImplementedError("write your pallas kernel here")


# speedup vs baseline: 2.4299x; 2.4299x over previous
import functools

import jax
import jax.numpy as jnp
from jax import lax
from jax.experimental import pallas as pl
from jax.experimental.pallas import tpu as pltpu
from jax.experimental.pallas import tpu_sc as plsc


def _mm(a, b):
    return jax.lax.dot_general(a, b, (((1,), (0,)), ((), ())),
                               preferred_element_type=jnp.float32)


def _proj_msg_body(nch, x_ref, winT_ref, bin_ref, wmsgT_ref, bmsg_ref,
                   state_ref, msg_ref):
    st = jnp.maximum(_mm(x_ref[...], winT_ref[...]) + bin_ref[...], 0.0)
    state_ref[...] = st
    m = jnp.maximum(_mm(st, wmsgT_ref[...]) + bmsg_ref[...], 0.0)
    cw = m.shape[1] // nch
    for c in range(nch):
        msg_ref[c] = m[:, c * cw:(c + 1) * cw]


def _gru_core(nch, h_dim, aggt_ref, state_ref, wihT_ref, whhT_ref, bih_ref,
              bhh_ref):
    agg = jnp.concatenate([aggt_ref[c] for c in range(nch)], axis=1)
    gi = _mm(agg, wihT_ref[...]) + bih_ref[...]
    gh = _mm(state_ref[...], whhT_ref[...]) + bhh_ref[...]
    i_r, i_z, i_n = (gi[:, :h_dim], gi[:, h_dim:2 * h_dim], gi[:, 2 * h_dim:])
    h_r, h_z, h_n = (gh[:, :h_dim], gh[:, h_dim:2 * h_dim], gh[:, 2 * h_dim:])
    r = jax.nn.sigmoid(i_r + h_r)
    z = jax.nn.sigmoid(i_z + h_z)
    n = jnp.tanh(i_n + r * h_n)
    return (1.0 - z) * n + z * state_ref[...]


def _gru_msg_body(nch, h_dim, aggt_ref, state_ref, wihT_ref, whhT_ref,
                  bih_ref, bhh_ref, wmsgT_ref, bmsg_ref,
                  newstate_ref, msg_ref):
    h = _gru_core(nch, h_dim, aggt_ref, state_ref, wihT_ref, whhT_ref,
                  bih_ref, bhh_ref)
    newstate_ref[...] = h
    m = jnp.maximum(_mm(h, wmsgT_ref[...]) + bmsg_ref[...], 0.0)
    cw = m.shape[1] // nch
    for c in range(nch):
        msg_ref[c] = m[:, c * cw:(c + 1) * cw]


def _gru_out_body(nch, h_dim, l_dim, aggt_ref, state_ref, wihT_ref, whhT_ref,
                  bih_ref, bhh_ref, woutT_ref, bout_ref, mu_ref, lv_ref):
    h = _gru_core(nch, h_dim, aggt_ref, state_ref, wihT_ref, whhT_ref,
                  bih_ref, bhh_ref)
    out = _mm(h, woutT_ref[...]) + bout_ref[...]
    mu_ref[...] = out[:, :l_dim]
    lv_ref[...] = out[:, l_dim:]


def _row_spec(bn, width):
    return pl.BlockSpec((bn, width), lambda i: (i, 0))


def _chunk_spec(nch, bn, cw):
    return pl.BlockSpec((nch, bn, cw), lambda i: (0, i, 0))


def _full_spec(shape):
    nd = len(shape)
    return pl.BlockSpec(shape, lambda i: (0,) * nd)


def _make_sc_scatter(n_nodes, e_pad, nch, cw, k_batch, num_cores,
                     num_subcores):
    n_sp = (n_nodes + 16 + 8 * num_subcores - 1) // (8 * num_subcores) \
        * (8 * num_subcores)
    zrows = n_sp // num_subcores
    eps = e_pad // num_subcores
    nbatch = eps // k_batch
    assert nbatch % 2 == 0
    chunks_per_core = nch // num_cores
    mesh = plsc.VectorSubcoreMesh(core_axis_name="c", subcore_axis_name="s")

    @functools.partial(
        pl.kernel,
        out_type=jax.ShapeDtypeStruct((nch, n_sp, cw), jnp.float32),
        mesh=mesh,
        scratch_types=[
            pltpu.VMEM((2, k_batch), jnp.int32),
            pltpu.VMEM((2, k_batch), jnp.int32),
            pltpu.VMEM((2, k_batch, cw), jnp.float32),
            pltpu.VMEM_SHARED((n_sp, cw), jnp.float32),
            pltpu.SemaphoreType.DMA,
            pltpu.SemaphoreType.DMA,
        ],
    )
    def sc_scatter(msg_hbm, src_hbm, dst_hbm, zeros_hbm, out_hbm,
                   src_v, dst_v, rows_v, agg_sh, sem0, sem1):
        core = lax.axis_index("c")
        sub = lax.axis_index("s")
        sems = (sem0, sem1)
        ebase = sub * eps

        for p in range(chunks_per_core):
            chunk = core * chunks_per_core + p
            tbl = msg_hbm.at[chunk]

            pltpu.sync_copy(zeros_hbm.at[pl.ds(sub * zrows, zrows)],
                            agg_sh.at[pl.ds(sub * zrows, zrows)])
            plsc.subcore_barrier()

            def issue(i, b):
                base = ebase + i * k_batch
                pltpu.sync_copy(src_hbm.at[pl.ds(base, k_batch)],
                                src_v.at[b])
                pltpu.sync_copy(dst_hbm.at[pl.ds(base, k_batch)],
                                dst_v.at[b])
                pltpu.async_copy(tbl.at[src_v.at[b]], rows_v.at[b], sems[b])

            def wait(b):
                pltpu.make_async_copy(zeros_hbm.at[pl.ds(0, k_batch)],
                                      rows_v.at[b], sems[b]).wait()

            def scatter(b):
                pltpu.sync_copy(rows_v.at[b], agg_sh.at[dst_v.at[b]],
                                add=True)

            issue(0, 0)

            def step(j, _):
                i = 2 * j
                issue(i + 1, 1)
                wait(0)
                scatter(0)
                issue(i + 2, 0)
                wait(1)
                scatter(1)
                return _

            lax.fori_loop(0, nbatch // 2 - 1, step, 0)
            i_last = nbatch - 2
            issue(i_last + 1, 1)
            wait(0)
            scatter(0)
            wait(1)
            scatter(1)

            plsc.subcore_barrier()
            pltpu.sync_copy(agg_sh.at[pl.ds(sub * zrows, zrows)],
                            out_hbm.at[chunk].at[pl.ds(sub * zrows, zrows)])
            plsc.subcore_barrier()

    return sc_scatter


def kernel(x, edge_index, W_in, b_in, W_msg, b_msg, W_ih, W_hh, b_ih, b_hh,
           W_mu, b_mu, W_lv, b_lv):
    n, f_dim = x.shape
    h_dim = W_in.shape[0]
    n_rounds = W_msg.shape[0]
    l_dim = W_mu.shape[0]
    e = edge_index.shape[1]

    nch = 4
    cw = h_dim // nch
    bn = 1000 if n % 1000 == 0 else n
    grid = (n // bn,)
    k_batch = 128
    num_cores, num_subcores = 2, 16

    src = edge_index[0].astype(jnp.int32)
    dst = edge_index[1].astype(jnp.int32)
    unit = 2 * num_subcores * k_batch
    e_pad = (e + unit - 1) // unit * unit
    n_sp = (n + 16 + 8 * num_subcores - 1) // (8 * num_subcores) \
        * (8 * num_subcores)
    src = jnp.pad(src, (0, e_pad - e))
    dst = jnp.pad(dst, (0, e_pad - e), constant_values=n)
    zeros = jnp.zeros((n_sp, cw), jnp.float32)

    winT = W_in.T
    wmsgT = jnp.transpose(W_msg, (0, 2, 1))
    wihT = W_ih.T
    whhT = W_hh.T
    bin2 = b_in.reshape(1, h_dim)
    bmsg2 = b_msg.reshape(n_rounds, 1, h_dim)
    bih2 = b_ih.reshape(1, 3 * h_dim)
    bhh2 = b_hh.reshape(1, 3 * h_dim)
    woutT = jnp.concatenate([W_mu.T, W_lv.T], axis=1)
    bout2 = jnp.concatenate([b_mu, b_lv]).reshape(1, 2 * l_dim)

    f32 = jnp.float32
    state_sds = jax.ShapeDtypeStruct((n, h_dim), f32)
    msg_sds = jax.ShapeDtypeStruct((nch, n, cw), f32)

    proj_call = pl.pallas_call(
        functools.partial(_proj_msg_body, nch),
        grid=grid,
        in_specs=[_row_spec(bn, f_dim), _full_spec((f_dim, h_dim)),
                  _full_spec((1, h_dim)), _full_spec((h_dim, h_dim)),
                  _full_spec((1, h_dim))],
        out_specs=[_row_spec(bn, h_dim), _chunk_spec(nch, bn, cw)],
        out_shape=[state_sds, msg_sds],
    )

    gru_msg_call = pl.pallas_call(
        functools.partial(_gru_msg_body, nch, h_dim),
        grid=grid,
        in_specs=[_chunk_spec(nch, bn, cw), _row_spec(bn, h_dim),
                  _full_spec((h_dim, 3 * h_dim)),
                  _full_spec((h_dim, 3 * h_dim)),
                  _full_spec((1, 3 * h_dim)), _full_spec((1, 3 * h_dim)),
                  _full_spec((h_dim, h_dim)), _full_spec((1, h_dim))],
        out_specs=[_row_spec(bn, h_dim), _chunk_spec(nch, bn, cw)],
        out_shape=[state_sds, msg_sds],
    )

    gru_out_call = pl.pallas_call(
        functools.partial(_gru_out_body, nch, h_dim, l_dim),
        grid=grid,
        in_specs=[_chunk_spec(nch, bn, cw), _row_spec(bn, h_dim),
                  _full_spec((h_dim, 3 * h_dim)),
                  _full_spec((h_dim, 3 * h_dim)),
                  _full_spec((1, 3 * h_dim)), _full_spec((1, 3 * h_dim)),
                  _full_spec((h_dim, 2 * l_dim)), _full_spec((1, 2 * l_dim))],
        out_specs=[_row_spec(bn, l_dim), _row_spec(bn, l_dim)],
        out_shape=[jax.ShapeDtypeStruct((n, l_dim), f32),
                   jax.ShapeDtypeStruct((n, l_dim), f32)],
    )

    sc_scatter = _make_sc_scatter(n, e_pad, nch, cw, k_batch, num_cores,
                                  num_subcores)

    state, msg_t = proj_call(x, winT, bin2, wmsgT[0], bmsg2[0])
    for r in range(n_rounds):
        agg_t = sc_scatter(msg_t, src, dst, zeros)
        if r < n_rounds - 1:
            state, msg_t = gru_msg_call(agg_t, state, wihT, whhT, bih2, bhh2,
                                        wmsgT[r + 1], bmsg2[r + 1])
        else:
            mu, lv = gru_out_call(agg_t, state, wihT, whhT, bih2, bhh2,
                                  woutT, bout2)
    return (mu, lv)
